# baseline (device time: 21492 ns/iter reference)
import jax
import jax.numpy as jnp
from jax import lax
from jax.experimental import pallas as pl
from jax.experimental.pallas import tpu as pltpu

M = 2048
N = 1024
HALF = N // 2
M_HALF = M // 2
C = 8
CH = M_HALF // C


def kernel(x):
    x = pltpu.with_memory_space_constraint(x, pltpu.MemorySpace.HBM)

    def body(
        x_ref,
        out_ref,
        chunk_f32,
        xsend_buf,
        xrecv_buf,
        res_buf,
        in_sems,
        out_sems,
        xsend_sems,
        xrecv_sems,
        ysend_sems,
        yrecv_sems,
    ):
        my_x = lax.axis_index("x")
        my_y = lax.axis_index("y")
        other_x = 1 - my_x
        other_y = 1 - my_y

        row0 = my_y * M_HALF

        dma_in = []
        for c in range(C):
            r = row0 + c * CH
            di = pltpu.make_async_copy(
                x_ref.at[0, pl.ds(r, CH), :],
                chunk_f32.at[c],
                in_sems.at[c],
            )
            di.start()
            dma_in.append(di)

        barrier_sem = pltpu.get_barrier_semaphore()
        pl.semaphore_signal(
            barrier_sem, inc=1,
            device_id=(other_x, my_y), device_id_type=pl.DeviceIdType.MESH,
        )
        pl.semaphore_signal(
            barrier_sem, inc=1,
            device_id=(my_x, other_y), device_id_type=pl.DeviceIdType.MESH,
        )
        pl.semaphore_wait(barrier_sem, 2)

        rdma_x = []
        for c in range(C):
            dma_in[c].wait()
            xsend_buf[c, :, :] = chunk_f32[
                c, :, pl.ds(other_x * HALF, HALF)
            ].astype(jnp.bfloat16)
            rx = pltpu.make_async_remote_copy(
                src_ref=xsend_buf.at[c],
                dst_ref=xrecv_buf.at[c],
                send_sem=xsend_sems.at[c],
                recv_sem=xrecv_sems.at[c],
                device_id=(other_x, my_y),
                device_id_type=pl.DeviceIdType.MESH,
            )
            rx.start()
            rdma_x.append(rx)

        for c in range(C):
            res_buf[c, :, :] = chunk_f32[
                c, :, pl.ds(my_x * HALF, HALF)
            ].astype(jnp.bfloat16)

        rdma_y = []
        dma_out = []
        for c in range(C):
            rdma_x[c].wait_recv()
            r = row0 + c * CH
            res_buf[c, :, :] = res_buf[c, :, :] + xrecv_buf[c, :, :]
            do = pltpu.make_async_copy(
                res_buf.at[c],
                out_ref.at[pl.ds(r, CH)],
                out_sems.at[c],
            )
            do.start()
            dma_out.append(do)
            ry = pltpu.make_async_remote_copy(
                src_ref=res_buf.at[c],
                dst_ref=out_ref.at[pl.ds(r, CH)],
                send_sem=ysend_sems.at[c],
                recv_sem=yrecv_sems.at[c],
                device_id=(my_x, other_y),
                device_id_type=pl.DeviceIdType.MESH,
            )
            ry.start()
            rdma_y.append(ry)

        for c in range(C):
            rdma_y[c].wait_recv()
        for c in range(C):
            dma_out[c].wait()
            rdma_y[c].wait_send()
            rdma_x[c].wait_send()

    return pl.pallas_call(
        body,
        out_shape=jax.ShapeDtypeStruct((M, HALF), jnp.bfloat16),
        in_specs=[pl.BlockSpec(memory_space=pltpu.MemorySpace.HBM)],
        out_specs=pl.BlockSpec(memory_space=pl.ANY),
        scratch_shapes=[
            pltpu.VMEM((C, CH, N), jnp.float32),
            pltpu.VMEM((C, CH, HALF), jnp.bfloat16),
            pltpu.VMEM((C, CH, HALF), jnp.bfloat16),
            pltpu.VMEM((C, CH, HALF), jnp.bfloat16),
            pltpu.SemaphoreType.DMA((C,)),
            pltpu.SemaphoreType.DMA((C,)),
            pltpu.SemaphoreType.DMA((C,)),
            pltpu.SemaphoreType.DMA((C,)),
            pltpu.SemaphoreType.DMA((C,)),
            pltpu.SemaphoreType.DMA((C,)),
        ],
        compiler_params=pltpu.CompilerParams(collective_id=0),
    )(x)


# device time: 21034 ns/iter; 1.0218x vs baseline; 1.0218x over previous
import jax
import jax.numpy as jnp
from jax import lax
from jax.experimental import pallas as pl
from jax.experimental.pallas import tpu as pltpu

M = 2048
N = 1024
HALF = N // 2
M_HALF = M // 2
C = 16
CH = M_HALF // C


def kernel(x):
    x = pltpu.with_memory_space_constraint(x, pltpu.MemorySpace.HBM)

    def body(
        x_ref,
        out_ref,
        chunk_f32,
        xsend_buf,
        xrecv_buf,
        res_buf,
        in_sems,
        out_sems,
        xsend_sems,
        xrecv_sems,
        ysend_sems,
        yrecv_sems,
    ):
        my_x = lax.axis_index("x")
        my_y = lax.axis_index("y")
        other_x = 1 - my_x
        other_y = 1 - my_y

        row0 = my_y * M_HALF

        dma_in = []
        for c in range(C):
            r = row0 + c * CH
            di = pltpu.make_async_copy(
                x_ref.at[0, pl.ds(r, CH), :],
                chunk_f32.at[c],
                in_sems.at[c],
            )
            di.start()
            dma_in.append(di)

        barrier_sem = pltpu.get_barrier_semaphore()
        pl.semaphore_signal(
            barrier_sem, inc=1,
            device_id=(other_x, my_y), device_id_type=pl.DeviceIdType.MESH,
        )
        pl.semaphore_signal(
            barrier_sem, inc=1,
            device_id=(my_x, other_y), device_id_type=pl.DeviceIdType.MESH,
        )
        pl.semaphore_wait(barrier_sem, 2)

        rdma_x = []
        for c in range(C):
            dma_in[c].wait()
            xsend_buf[c, :, :] = chunk_f32[
                c, :, pl.ds(other_x * HALF, HALF)
            ].astype(jnp.bfloat16)
            rx = pltpu.make_async_remote_copy(
                src_ref=xsend_buf.at[c],
                dst_ref=xrecv_buf.at[c],
                send_sem=xsend_sems.at[c],
                recv_sem=xrecv_sems.at[c],
                device_id=(other_x, my_y),
                device_id_type=pl.DeviceIdType.MESH,
            )
            rx.start()
            rdma_x.append(rx)

        for c in range(C):
            res_buf[c, :, :] = chunk_f32[
                c, :, pl.ds(my_x * HALF, HALF)
            ].astype(jnp.bfloat16)

        rdma_y = []
        dma_out = []
        for c in range(C):
            rdma_x[c].wait_recv()
            r = row0 + c * CH
            res_buf[c, :, :] = res_buf[c, :, :] + xrecv_buf[c, :, :]
            do = pltpu.make_async_copy(
                res_buf.at[c],
                out_ref.at[pl.ds(r, CH)],
                out_sems.at[c],
            )
            do.start()
            dma_out.append(do)
            ry = pltpu.make_async_remote_copy(
                src_ref=res_buf.at[c],
                dst_ref=out_ref.at[pl.ds(r, CH)],
                send_sem=ysend_sems.at[c],
                recv_sem=yrecv_sems.at[c],
                device_id=(my_x, other_y),
                device_id_type=pl.DeviceIdType.MESH,
            )
            ry.start()
            rdma_y.append(ry)

        for c in range(C):
            rdma_y[c].wait_recv()
        for c in range(C):
            dma_out[c].wait()
            rdma_y[c].wait_send()
            rdma_x[c].wait_send()

    return pl.pallas_call(
        body,
        out_shape=jax.ShapeDtypeStruct((M, HALF), jnp.bfloat16),
        in_specs=[pl.BlockSpec(memory_space=pltpu.MemorySpace.HBM)],
        out_specs=pl.BlockSpec(memory_space=pl.ANY),
        scratch_shapes=[
            pltpu.VMEM((C, CH, N), jnp.float32),
            pltpu.VMEM((C, CH, HALF), jnp.bfloat16),
            pltpu.VMEM((C, CH, HALF), jnp.bfloat16),
            pltpu.VMEM((C, CH, HALF), jnp.bfloat16),
            pltpu.SemaphoreType.DMA((C,)),
            pltpu.SemaphoreType.DMA((C,)),
            pltpu.SemaphoreType.DMA((C,)),
            pltpu.SemaphoreType.DMA((C,)),
            pltpu.SemaphoreType.DMA((C,)),
            pltpu.SemaphoreType.DMA((C,)),
        ],
        compiler_params=pltpu.CompilerParams(collective_id=0),
    )(x)
